# baseline (speedup 1.0000x reference)
import jax
import jax.numpy as jnp
from jax import lax
from jax.experimental import pallas as pl
from jax.experimental.pallas import tpu as pltpu
from jax.experimental.pallas import tpu_sc as plsc

IN_FEATURES = 1000000
OUT_FEATURES = 64
BATCH = 16384
NUM_FIELDS = 26

_INFO = plsc.get_sparse_core_info()
NC = _INFO.num_cores
NS = _INFO.num_subcores
NW = NC * NS
BPW = BATCH // NW
LANES = 16
CPD = OUT_FEATURES // LANES
RPG = 4
GL = RPG * NUM_FIELDS
NG = BPW // RPG
K = 4
OUTER = NG // K
SB = 128
OPS = SB // (RPG * K)


def _body(idx_hbm, w_hbm, bias_hbm, out_hbm, raw_v, rows_v, acc_v, bias_v,
          sem0, sem1, sem2, sem3, out_sem):
    sems = (sem0, sem1, sem2, sem3)
    wid = lax.axis_index("s") * NC + lax.axis_index("c")
    base = wid * BPW

    pltpu.sync_copy(bias_hbm, bias_v)
    pltpu.sync_copy(
        idx_hbm.at[pl.ds(base * NUM_FIELDS, BPW * NUM_FIELDS)], raw_v
    )

    bias_regs = [bias_v[pl.ds(c * LANES, LANES)] for c in range(CPD)]

    def fire(g, b):
        return pltpu.async_copy(
            w_hbm.at[raw_v.at[pl.ds(g * GL, GL)]],
            rows_v.at[b],
            sems[b],
        )

    for b in range(K):
        fire(b, b)

    def outer(o, c2):
        for b in range(K):
            g = o * K + b
            pltpu.make_async_copy(
                w_hbm.at[pl.ds(0, GL)], rows_v.at[b], sems[b]
            ).wait()

            for i in range(RPG):
                accs = list(bias_regs)
                for j in range(NUM_FIELDS):
                    for c in range(CPD):
                        accs[c] = accs[c] + rows_v[
                            b, i * NUM_FIELDS + j, pl.ds(c * LANES, LANES)
                        ]
                row = g * RPG + i
                for c in range(CPD):
                    acc_v[row, pl.ds(c * LANES, LANES)] = accs[c]

            @pl.when(o < OUTER - 1)
            def _():
                fire(g + K, b)

        @pl.when(lax.rem(o, OPS) == OPS - 1)
        def _():
            row0 = (o // OPS) * SB
            pltpu.async_copy(
                acc_v.at[pl.ds(row0, SB)],
                out_hbm.at[pl.ds(base + row0, SB)],
                out_sem,
            )

        return c2

    lax.fori_loop(0, OUTER, outer, 0)

    for _ in range(BPW // SB):
        pltpu.make_async_copy(
            w_hbm.at[pl.ds(0, SB)], acc_v.at[pl.ds(0, SB)], out_sem
        ).wait()


@jax.jit
def _run(idx_flat, weight, bias):
    kern = pl.kernel(
        _body,
        mesh=plsc.VectorSubcoreMesh(core_axis_name="c", subcore_axis_name="s"),
        compiler_params=pltpu.CompilerParams(use_tc_tiling_on_sc=False),
        out_type=jax.ShapeDtypeStruct((BATCH, OUT_FEATURES), jnp.float32),
        scratch_types=[
            pltpu.VMEM((BPW * NUM_FIELDS,), jnp.int32),
            pltpu.VMEM((K, GL, OUT_FEATURES), jnp.float32),
            pltpu.VMEM((BPW, OUT_FEATURES), jnp.float32),
            pltpu.VMEM((OUT_FEATURES,), jnp.float32),
            pltpu.SemaphoreType.DMA,
            pltpu.SemaphoreType.DMA,
            pltpu.SemaphoreType.DMA,
            pltpu.SemaphoreType.DMA,
            pltpu.SemaphoreType.DMA,
        ],
    )
    return kern(idx_flat, weight, bias)


def kernel(indices, weight, bias):
    idx_flat = jnp.asarray(indices, dtype=jnp.int32).reshape(-1)
    return _run(idx_flat, weight, bias)


# speedup vs baseline: 1.5687x; 1.5687x over previous
import jax
import jax.numpy as jnp
from jax import lax
from jax.experimental import pallas as pl
from jax.experimental.pallas import tpu as pltpu
from jax.experimental.pallas import tpu_sc as plsc

IN_FEATURES = 1000000
OUT_FEATURES = 64
BATCH = 16384
NUM_FIELDS = 26

_INFO = plsc.get_sparse_core_info()
NC = _INFO.num_cores
NS = _INFO.num_subcores
LANES = 16
CPSC = OUT_FEATURES // NC
BPT = BATCH // NS
NV = BPT // LANES
CHUNK = 62464
LCHUNK = 62976
TAIL_OFF = 15 * CHUNK + LCHUNK
TAIL = IN_FEATURES - TAIL_OFF


def _body(wt_hbm, idx_hbm, brep_hbm, out_hbm, row_sh, idxs_v,
          gath_v, outrow_v, brep_v, tail_v, lsem, gsem, osem, tsem):
    sc = lax.axis_index("c")
    tile = lax.axis_index("s")
    b0 = pl.multiple_of(tile * BPT, BPT)
    off = pl.multiple_of(tile * CHUNK, 1024)

    def fire_load(cc):
        c = sc * CPSC + cc

        @pl.when(tile != NS - 1)
        def _():
            pltpu.async_copy(
                wt_hbm.at[c, pl.ds(off, CHUNK)],
                row_sh.at[pl.ds(off, CHUNK)],
                lsem,
            )

        @pl.when(tile == NS - 1)
        def _():
            loff = pl.multiple_of((NS - 1) * CHUNK, 1024)
            pltpu.async_copy(
                wt_hbm.at[c, pl.ds(loff, LCHUNK)],
                row_sh.at[pl.ds(loff, LCHUNK)],
                lsem,
            )

        @pl.when(tile == 0)
        def _():
            c8 = pl.multiple_of((c // 8) * 8, 8)
            pltpu.async_copy(
                wt_hbm.at[pl.ds(c8, 8), pl.ds(TAIL_OFF, TAIL)], tail_v, tsem
            ).wait()
            pltpu.async_copy(
                tail_v.at[lax.rem(c, 8)],
                row_sh.at[pl.ds(TAIL_OFF, TAIL)],
                tsem,
            ).wait()

    def drain_load():
        @pl.when(tile != NS - 1)
        def _():
            pltpu.make_async_copy(
                wt_hbm.at[0, pl.ds(0, CHUNK)],
                row_sh.at[pl.ds(0, CHUNK)],
                lsem,
            ).wait()

        @pl.when(tile == NS - 1)
        def _():
            pltpu.make_async_copy(
                wt_hbm.at[0, pl.ds(0, LCHUNK)],
                row_sh.at[pl.ds(0, LCHUNK)],
                lsem,
            ).wait()

    fire_load(0)
    pltpu.sync_copy(brep_hbm, brep_v)
    stage = [
        pltpu.async_copy(
            idx_hbm.at[j, pl.ds(b0, BPT)],
            idxs_v.at[pl.ds(j * BPT, BPT)],
            gsem,
        )
        for j in range(NUM_FIELDS)
    ]
    for cp in stage:
        cp.wait()
    drain_load()
    plsc.subcore_barrier()

    for cc in range(CPSC):
        c = sc * CPSC + cc

        pltpu.async_copy(row_sh.at[idxs_v], gath_v, gsem).wait()
        plsc.subcore_barrier()
        if cc + 1 < CPSC:
            fire_load(cc + 1)

        bias_vec = brep_v[pl.ds(c * LANES, LANES)]

        @plsc.parallel_loop(0, NV)
        def _reduce(v):
            bo = v * LANES
            acc = bias_vec
            for j in range(NUM_FIELDS):
                acc = acc + gath_v[pl.ds(j * BPT + bo, LANES)]
            outrow_v[pl.ds(bo, LANES)] = acc

        pltpu.async_copy(
            outrow_v, out_hbm.at[c, pl.ds(b0, BPT)], osem
        ).wait()

        if cc + 1 < CPSC:
            drain_load()
            plsc.subcore_barrier()


@jax.jit
def _run(wt, idx_t, bias_rep):
    kern = pl.kernel(
        _body,
        mesh=plsc.VectorSubcoreMesh(core_axis_name="c", subcore_axis_name="s"),
        compiler_params=pltpu.CompilerParams(use_tc_tiling_on_sc=True),
        out_type=jax.ShapeDtypeStruct((OUT_FEATURES, BATCH), jnp.float32),
        scratch_types=[
            pltpu.VMEM_SHARED((IN_FEATURES,), jnp.float32),
            pltpu.VMEM((NUM_FIELDS * BPT,), jnp.int32),
            pltpu.VMEM((NUM_FIELDS * BPT,), jnp.float32),
            pltpu.VMEM((BPT,), jnp.float32),
            pltpu.VMEM((OUT_FEATURES * LANES,), jnp.float32),
            pltpu.VMEM((8, TAIL), jnp.float32),
            pltpu.SemaphoreType.DMA,
            pltpu.SemaphoreType.DMA,
            pltpu.SemaphoreType.DMA,
            pltpu.SemaphoreType.DMA,
        ],
    )
    return kern(wt, idx_t, bias_rep)


def kernel(indices, weight, bias):
    wt = weight.T
    idx_t = jnp.asarray(indices, dtype=jnp.int32).T
    idx_t = jnp.pad(idx_t, ((0, 32 - NUM_FIELDS), (0, 0)))
    bias_rep = jnp.repeat(bias, LANES)
    out_t = _run(wt, idx_t, bias_rep)
    return out_t.T
